# baseline (speedup 1.0000x reference)
import jax
import jax.numpy as jnp
from jax.experimental import pallas as pl
from jax.experimental.pallas import tpu as pltpu

N_EXPERTS = 64
K_TOP = 8
HIDDEN_DIM = 4096
N_GROUPS = 8
GSIZE = 8
TOPK_GROUPS = 4
ROUTE_SCALE = 2.5
TB = 512
KB = 1024
NK = HIDDEN_DIM // KB

_NEG = -1e30


def _router_block(hs_ref, wt_ref, bias_ref, idx_ref, w_ref, acc_ref):
    k = pl.program_id(1)
    part = jax.lax.dot_general(
        hs_ref[...], wt_ref[...],
        dimension_numbers=(((1,), (1,)), ((), ())),
        preferred_element_type=jnp.float32,
    )

    @pl.when(k == 0)
    def _init():
        acc_ref[...] = part

    @pl.when(k > 0)
    def _accum():
        acc_ref[...] += part

    @pl.when(k == NK - 1)
    def _route():
        _routing(acc_ref[...], bias_ref, idx_ref, w_ref)


def _routing(logits, bias_ref, idx_ref, w_ref):
    lt = logits.T
    scores = jax.nn.sigmoid(lt)
    sfc = scores + bias_ref[...]
    eidx = jax.lax.broadcasted_iota(
        jnp.int32, (N_EXPERTS, TB), 0).astype(jnp.float32)

    gparts = []
    for g in range(N_GROUPS):
        sg = sfc[g * GSIZE:(g + 1) * GSIZE, :]
        m1 = jnp.max(sg, axis=0, keepdims=True)
        m2 = jnp.max(jnp.where(sg == m1, _NEG, sg), axis=0, keepdims=True)
        gparts.append(m1 + m2)
    gs = jnp.concatenate(gparts, axis=0)

    gidx = jax.lax.broadcasted_iota(
        jnp.int32, (N_GROUPS, TB), 0).astype(jnp.float32)
    gmask = jnp.zeros((N_GROUPS, TB), jnp.float32)
    gwork = gs
    for _ in range(TOPK_GROUPS):
        gm = jnp.max(gwork, axis=0, keepdims=True)
        gl = jnp.min(jnp.where(gwork == gm, gidx, float(N_GROUPS)),
                     axis=0, keepdims=True)
        sel = gidx == gl
        gmask = jnp.where(sel, 1.0, gmask)
        gwork = jnp.where(sel, _NEG, gwork)

    smask = jnp.concatenate(
        [jnp.broadcast_to(gmask[g:g + 1, :], (GSIZE, TB))
         for g in range(N_GROUPS)], axis=0)
    ms = jnp.where(smask > 0.0, sfc, 0.0)

    idxs, ws = [], []
    for _ in range(K_TOP):
        m = jnp.max(ms, axis=0, keepdims=True)
        l = jnp.min(jnp.where(ms == m, eidx, float(N_EXPERTS)),
                    axis=0, keepdims=True)
        sel = eidx == l
        w = jnp.sum(jnp.where(sel, scores, 0.0), axis=0, keepdims=True)
        idxs.append(l)
        ws.append(w)
        ms = jnp.where(sel, _NEG, ms)
    tidx_t = jnp.concatenate(idxs, axis=0)
    tw_t = jnp.concatenate(ws, axis=0)
    tw_t = tw_t / (jnp.sum(tw_t, axis=0, keepdims=True) + 1e-20) * ROUTE_SCALE

    idx_ref[...] = tidx_t.T.astype(jnp.int32)
    w_ref[...] = tw_t.T


def kernel(hidden_states, weight, e_score_correction_bias):
    hs = hidden_states.reshape(-1, hidden_states.shape[-1]).astype(jnp.float32)
    tokens = hs.shape[0]
    wt = weight.astype(jnp.float32)
    bias = e_score_correction_bias.astype(jnp.float32).reshape(N_EXPERTS, 1)
    grid = (tokens // TB, NK)
    tidx, tw = pl.pallas_call(
        _router_block,
        grid=grid,
        in_specs=[
            pl.BlockSpec((TB, KB), lambda i, k: (i, k)),
            pl.BlockSpec((N_EXPERTS, KB), lambda i, k: (0, k)),
            pl.BlockSpec((N_EXPERTS, 1), lambda i, k: (0, 0)),
        ],
        out_specs=[
            pl.BlockSpec((TB, K_TOP), lambda i, k: (i, 0)),
            pl.BlockSpec((TB, K_TOP), lambda i, k: (i, 0)),
        ],
        out_shape=[
            jax.ShapeDtypeStruct((tokens, K_TOP), jnp.int32),
            jax.ShapeDtypeStruct((tokens, K_TOP), jnp.float32),
        ],
        scratch_shapes=[pltpu.VMEM((TB, N_EXPERTS), jnp.float32)],
    )(hs, wt, bias)
    return tidx, tw


# speedup vs baseline: 1.4859x; 1.4859x over previous
import functools

import jax
import jax.numpy as jnp
from jax import lax
from jax.experimental import pallas as pl
from jax.experimental.pallas import tpu as pltpu
from jax.experimental.pallas import tpu_sc as plsc

N_EXPERTS = 64
K_TOP = 8
HIDDEN_DIM = 4096
N_GROUPS = 8
GSIZE = 8
TOPK_GROUPS = 4
ROUTE_SCALE = 2.5
TB = 512

_NEG = -1e30

NW = 32
CHUNK = 16


def _scores_block(hs_ref, wt_ref, out_ref):
    logits = jax.lax.dot_general(
        hs_ref[...], wt_ref[...],
        dimension_numbers=(((1,), (1,)), ((), ())),
        preferred_element_type=jnp.float32,
    )
    out_ref[...] = jax.nn.sigmoid(logits.T)


def _tc_scores(hs, wt):
    tokens = hs.shape[0]
    return pl.pallas_call(
        _scores_block,
        grid=(tokens // TB,),
        in_specs=[
            pl.BlockSpec((TB, HIDDEN_DIM), lambda i: (i, 0)),
            pl.BlockSpec((N_EXPERTS, HIDDEN_DIM), lambda i: (0, 0)),
        ],
        out_specs=pl.BlockSpec((N_EXPERTS, TB), lambda i: (0, i)),
        out_shape=jax.ShapeDtypeStruct((N_EXPERTS, tokens), jnp.float32),
    )(hs, wt)


def _sc_route_body(scores_hbm, biasx_hbm, idx_hbm, w_hbm,
                   stile, btile, oidx, ow):
    tokens_per_w = stile.shape[1]
    wid = lax.axis_index("s") * 2 + lax.axis_index("c")
    base = wid * tokens_per_w
    pltpu.sync_copy(scores_hbm.at[:, pl.ds(base, tokens_per_w)], stile)
    pltpu.sync_copy(biasx_hbm, btile)

    def chunk_body(c, carry):
        col = c * CHUNK
        tloc = c * CHUNK + lax.iota(jnp.int32, CHUNK)

        gmaxes = []
        for g in range(N_GROUPS):
            m1 = None
            m2 = jnp.full((CHUNK,), _NEG, jnp.float32)
            for j in range(GSIZE):
                e = g * GSIZE + j
                sfc = stile[e, pl.ds(col, CHUNK)] + btile[e, :]
                if m1 is None:
                    m1 = sfc
                else:
                    hi = jnp.maximum(m1, sfc)
                    m2 = jnp.maximum(m2, jnp.minimum(m1, sfc))
                    m1 = hi
            gmaxes.append(m1 + m2)

        t = [jnp.full((CHUNK,), _NEG, jnp.float32) for _ in range(TOPK_GROUPS)]
        for g in range(N_GROUPS):
            cva = gmaxes[g]
            for j in range(TOPK_GROUPS):
                hi = jnp.maximum(t[j], cva)
                cva = jnp.minimum(t[j], cva)
                t[j] = hi
        thr = t[TOPK_GROUPS - 1]
        gmask = [gmaxes[g] >= thr for g in range(N_GROUPS)]

        wv = [jnp.full((CHUNK,), _NEG, jnp.float32) for _ in range(K_TOP)]
        sv = [jnp.zeros((CHUNK,), jnp.float32) for _ in range(K_TOP)]
        iv = [jnp.zeros((CHUNK,), jnp.int32) for _ in range(K_TOP)]
        for e in range(N_EXPERTS):
            raw = stile[e, pl.ds(col, CHUNK)]
            sfc = raw + btile[e, :]
            cv = jnp.where(gmask[e // GSIZE], sfc, 0.0)
            cs = raw
            ci = jnp.full((CHUNK,), e, jnp.int32)
            depth = min(e + 1, K_TOP)
            for j in range(depth):
                cond = cv > wv[j]
                nwv = jnp.where(cond, cv, wv[j])
                cv = jnp.where(cond, wv[j], cv)
                nsv = jnp.where(cond, cs, sv[j])
                cs = jnp.where(cond, sv[j], cs)
                niv = jnp.where(cond, ci, iv[j])
                ci = jnp.where(cond, iv[j], ci)
                wv[j] = nwv
                sv[j] = nsv
                iv[j] = niv

        ssum = sv[0]
        for j in range(1, K_TOP):
            ssum = ssum + sv[j]
        for j in range(K_TOP):
            oidx[j, pl.ds(col, CHUNK)] = iv[j]
            ow[j, pl.ds(col, CHUNK)] = sv[j] / (ssum + 1e-20) * ROUTE_SCALE
        return carry

    lax.fori_loop(0, tokens_per_w // CHUNK, chunk_body, 0)

    pltpu.sync_copy(oidx, idx_hbm.at[:, pl.ds(base, tokens_per_w)])
    pltpu.sync_copy(ow, w_hbm.at[:, pl.ds(base, tokens_per_w)])


def _sc_route(scores_t, bias_x, tokens):
    tokens_per_w = tokens // NW
    mesh = plsc.VectorSubcoreMesh(core_axis_name="c", subcore_axis_name="s")
    fn = functools.partial(
        pl.kernel,
        mesh=mesh,
        out_type=[
            jax.ShapeDtypeStruct((K_TOP, tokens), jnp.int32),
            jax.ShapeDtypeStruct((K_TOP, tokens), jnp.float32),
        ],
        scratch_types=[
            pltpu.VMEM((N_EXPERTS, tokens_per_w), jnp.float32),
            pltpu.VMEM((N_EXPERTS, CHUNK), jnp.float32),
            pltpu.VMEM((K_TOP, tokens_per_w), jnp.int32),
            pltpu.VMEM((K_TOP, tokens_per_w), jnp.float32),
        ],
    )(_sc_route_body)
    return fn(scores_t, bias_x)


def kernel(hidden_states, weight, e_score_correction_bias):
    hs = hidden_states.reshape(-1, hidden_states.shape[-1]).astype(jnp.float32)
    tokens = hs.shape[0]
    wt = weight.astype(jnp.float32)
    bias = e_score_correction_bias.astype(jnp.float32)
    bias_x = jnp.broadcast_to(bias.reshape(N_EXPERTS, 1),
                              (N_EXPERTS, CHUNK))
    scores_t = _tc_scores(hs, wt)
    tidx_t, tw_t = _sc_route(scores_t, bias_x, tokens)
    return tidx_t.T, tw_t.T
